# Initial kernel scaffold; baseline (speedup 1.0000x reference)
import jax
import jax.numpy as jnp
from jax.experimental import pallas as pl


def kernel(input_ids, embed_weight):
    raise NotImplementedError("write your pallas kernel here")


# speedup vs baseline: 13.1390x; 13.1390x over previous
import functools

import jax
import jax.numpy as jnp
from jax import lax
from jax.experimental import pallas as pl
from jax.experimental.pallas import tpu as pltpu
from jax.experimental.pallas import tpu_sc as plsc

VOCAB = 1000000
EMBED_DIM = 4
BATCH = 16384
SEQ = 200

NC = 2
NS = 16
NW = NC * NS

DP = 8
CHUNK = 128
N_TOTAL = BATCH * SEQ
PER_TILE = N_TOTAL // NW
N_CHUNKS = PER_TILE // CHUNK
K = 8
N_OUTER = N_CHUNKS // K
OUT_W = CHUNK * EMBED_DIM


def _gather_body(table_hbm, ids_hbm, out_hbm, idx_blk, rows_buf, out_buf, sem):
    wid = lax.axis_index("s") * NC + lax.axis_index("c")
    iota = lax.iota(jnp.int32, 16)
    row_pat = jnp.right_shift(iota, 2)
    col_pat = jnp.bitwise_and(iota, 3)

    def outer(g, _):
        pltpu.sync_copy(ids_hbm.at[wid, pl.ds(g * K, K)], idx_blk)
        copies = [
            pltpu.async_copy(table_hbm.at[idx_blk.at[k]], rows_buf.at[k], sem)
            for k in range(K)
        ]
        for k in range(K):
            copies[k].wait()
            ob = out_buf.at[k]
            for v in range(OUT_W // 16):
                vals = plsc.load_gather(
                    rows_buf.at[k], [4 * v + row_pat, col_pat])
                ob[pl.ds(16 * v, 16)] = vals
        pltpu.sync_copy(out_buf, out_hbm.at[wid, pl.ds(g * K, K)])
        return _

    lax.fori_loop(0, N_OUTER, outer, 0)


@jax.jit
def _embed_gather(ids_flat, table_pad):
    mesh = plsc.VectorSubcoreMesh(core_axis_name="c", subcore_axis_name="s",
                                  num_cores=NC, num_subcores=NS)
    f = pl.kernel(
        _gather_body,
        out_type=jax.ShapeDtypeStruct((NW, N_CHUNKS, OUT_W), jnp.float32),
        mesh=mesh,
        scratch_types=[
            pltpu.VMEM((K, CHUNK), jnp.int32),
            pltpu.VMEM((K, CHUNK, DP), jnp.float32),
            pltpu.VMEM((K, OUT_W), jnp.float32),
            pltpu.SemaphoreType.DMA,
        ],
        compiler_params=pltpu.CompilerParams(use_tc_tiling_on_sc=False,
                                             needs_layout_passes=False),
    )
    return f(table_pad, ids_flat)


def kernel(input_ids, embed_weight):
    ids = input_ids.astype(jnp.int32).reshape(NW, N_CHUNKS, CHUNK)
    table_pad = jnp.pad(embed_weight, ((0, 0), (0, DP - EMBED_DIM)))
    out = _embed_gather(ids, table_pad)
    return out.reshape(BATCH, SEQ, EMBED_DIM)
